# baseline (speedup 1.0000x reference)
import functools

import jax
import jax.numpy as jnp
from jax import lax
from jax.experimental import pallas as pl
from jax.experimental.pallas import tpu as pltpu
from jax.experimental.pallas import tpu_sc as plsc

EMBED = 128
NUM_WORKERS = 32
TOKENS = 4096 * 200
X_ROWS = TOKENS // 128
ROWS_PER_WORKER = X_ROWS // NUM_WORKERS
R = 2
NCHUNK = ROWS_PER_WORKER // R


def _proj_body(e0, e1, e2, e3, w0, w1, w2, w3, o0, o1, o2, o3):
    o0[...] = jnp.dot(e0[...], w0[...], preferred_element_type=jnp.float32)
    o1[...] = jnp.dot(e1[...], w1[...], preferred_element_type=jnp.float32)
    o2[...] = jnp.dot(e2[...], w2[...], preferred_element_type=jnp.float32)
    o3[...] = jnp.dot(e3[...], w3[...], preferred_element_type=jnp.float32)


def _project_tables(embs, ws):
    outs = pl.pallas_call(
        _proj_body,
        out_shape=[jax.ShapeDtypeStruct((e.shape[0], EMBED), jnp.float32)
                   for e in embs],
    )(*embs, *ws)
    return jnp.concatenate(outs, axis=0)


NPAIR = NCHUNK // 2


def _sc_body(x_hbm, tab_hbm, out_hbm, tab_sh, idx_v,
             rows_v, gsem0, gsem1, ssem0, ssem1, hsem):
    sid = lax.axis_index("s")
    wid = sid * 2 + lax.axis_index("c")
    row0 = wid * ROWS_PER_WORKER
    gsems = (gsem0, gsem1)
    ssems = (ssem0, ssem1)

    @pl.when(sid == 0)
    def _():
        pltpu.sync_copy(tab_hbm, tab_sh)

    pltpu.sync_copy(x_hbm.at[pl.ds(row0, ROWS_PER_WORKER)], idx_v)
    plsc.subcore_barrier()

    def gissue(jj, b, srcs):
        for t in range(R):
            sem = hsem if srcs[t] is tab_hbm else gsems[b]
            pltpu.async_copy(srcs[t].at[idx_v.at[jj * R + t]],
                             rows_v.at[b * R + t], sem)

    def gwait(b, srcs):
        for t in range(R):
            sem = hsem if srcs[t] is tab_hbm else gsems[b]
            pltpu.make_async_copy(srcs[t].at[idx_v.at[b * R + t]],
                                  rows_v.at[b * R + t], sem).wait()

    def sissue(jj, b):
        pltpu.async_copy(rows_v.at[pl.ds(b * R, R)],
                         out_hbm.at[pl.ds(row0 + jj * R, R)], ssems[b])

    def swait(b):
        pltpu.make_async_copy(rows_v.at[pl.ds(b * R, R)],
                              out_hbm.at[pl.ds(row0, R)], ssems[b]).wait()

    gissue(0, 0, (tab_sh, tab_sh))

    def pair(g, carry):
        jj0 = 2 * g
        gwait(0, (tab_sh, tab_sh))
        sissue(jj0, 0)

        @pl.when(g > 0)
        def _():
            swait(1)

        gissue(jj0 + 1, 1, (tab_sh, tab_hbm))

        gwait(1, (tab_sh, tab_hbm))
        sissue(jj0 + 1, 1)
        swait(0)

        @pl.when(g < NPAIR - 1)
        def _():
            gissue(jj0 + 2, 0, (tab_sh, tab_sh))

        return carry

    lax.fori_loop(0, NPAIR, pair, 0)
    swait(1)


def kernel(x, emb0, emb1, emb2, emb3, W0, W1, W2, W3):
    table = _project_tables([emb0, emb1, emb2, emb3], [W0, W1, W2, W3])
    x2d = x.reshape(X_ROWS, 128)

    mesh = plsc.VectorSubcoreMesh(core_axis_name="c", subcore_axis_name="s")
    gather = functools.partial(
        pl.kernel,
        mesh=mesh,
        out_type=jax.ShapeDtypeStruct((X_ROWS, 128, EMBED), jnp.float32),
        scratch_types=[
            pltpu.VMEM_SHARED((1000, EMBED), jnp.float32),
            pltpu.VMEM((ROWS_PER_WORKER, 128), jnp.int32),
            pltpu.VMEM((2 * R, 128, EMBED), jnp.float32),
            pltpu.SemaphoreType.DMA,
            pltpu.SemaphoreType.DMA,
            pltpu.SemaphoreType.DMA,
            pltpu.SemaphoreType.DMA,
            pltpu.SemaphoreType.DMA,
        ],
    )(_sc_body)
    out = gather(x2d, table)
    return out.reshape(x.shape + (EMBED,))


# speedup vs baseline: 1.0395x; 1.0395x over previous
import functools

import jax
import jax.numpy as jnp
from jax import lax
from jax.experimental import pallas as pl
from jax.experimental.pallas import tpu as pltpu
from jax.experimental.pallas import tpu_sc as plsc

EMBED = 128
NUM_WORKERS = 32
TOKENS = 4096 * 200
X_ROWS = TOKENS // 128
ROWS_PER_WORKER = X_ROWS // NUM_WORKERS
R = 2
NCHUNK = ROWS_PER_WORKER // R


def _proj_body(e0, e1, e2, e3, w0, w1, w2, w3, o):
    o[0:100] = jnp.dot(e0[...], w0[...], preferred_element_type=jnp.float32)
    o[100:300] = jnp.dot(e1[...], w1[...], preferred_element_type=jnp.float32)
    o[300:600] = jnp.dot(e2[...], w2[...], preferred_element_type=jnp.float32)
    o[600:1000] = jnp.dot(e3[...], w3[...], preferred_element_type=jnp.float32)


def _project_tables(embs, ws):
    return pl.pallas_call(
        _proj_body,
        out_shape=jax.ShapeDtypeStruct((1000, EMBED), jnp.float32),
    )(*embs, *ws)


NPAIR = NCHUNK // 2


def _sc_body(x_hbm, tab_hbm, out_hbm, tab_sh, idx_v,
             rows_v, gsem0, gsem1, ssem0, ssem1, hsem):
    sid = lax.axis_index("s")
    wid = sid * 2 + lax.axis_index("c")
    row0 = wid * ROWS_PER_WORKER
    gsems = (gsem0, gsem1)
    ssems = (ssem0, ssem1)

    @pl.when(sid == 0)
    def _():
        pltpu.sync_copy(tab_hbm, tab_sh)

    pltpu.sync_copy(x_hbm.at[pl.ds(row0, ROWS_PER_WORKER)], idx_v)
    plsc.subcore_barrier()

    def gissue(jj, b, srcs):
        for t in range(R):
            sem = hsem if srcs[t] is tab_hbm else gsems[b]
            pltpu.async_copy(srcs[t].at[idx_v.at[jj * R + t]],
                             rows_v.at[b * R + t], sem)

    def gwait(b, srcs):
        for t in range(R):
            sem = hsem if srcs[t] is tab_hbm else gsems[b]
            pltpu.make_async_copy(srcs[t].at[idx_v.at[b * R + t]],
                                  rows_v.at[b * R + t], sem).wait()

    def sissue(jj, b):
        pltpu.async_copy(rows_v.at[pl.ds(b * R, R)],
                         out_hbm.at[pl.ds(row0 + jj * R, R)], ssems[b])

    def swait(b):
        pltpu.make_async_copy(rows_v.at[pl.ds(b * R, R)],
                              out_hbm.at[pl.ds(row0, R)], ssems[b]).wait()

    gissue(0, 0, (tab_sh, tab_sh))

    def pair(g, carry):
        jj0 = 2 * g
        gwait(0, (tab_sh, tab_sh))
        sissue(jj0, 0)

        @pl.when(g > 0)
        def _():
            swait(1)

        gissue(jj0 + 1, 1, (tab_sh, tab_hbm))

        gwait(1, (tab_sh, tab_hbm))
        sissue(jj0 + 1, 1)
        swait(0)

        @pl.when(g < NPAIR - 1)
        def _():
            gissue(jj0 + 2, 0, (tab_sh, tab_sh))

        return carry

    lax.fori_loop(0, NPAIR, pair, 0)
    swait(1)


def kernel(x, emb0, emb1, emb2, emb3, W0, W1, W2, W3):
    table = _project_tables([emb0, emb1, emb2, emb3], [W0, W1, W2, W3])
    x2d = x.reshape(X_ROWS, 128)

    mesh = plsc.VectorSubcoreMesh(core_axis_name="c", subcore_axis_name="s")
    gather = functools.partial(
        pl.kernel,
        mesh=mesh,
        out_type=jax.ShapeDtypeStruct((X_ROWS, 128, EMBED), jnp.float32),
        scratch_types=[
            pltpu.VMEM_SHARED((1000, EMBED), jnp.float32),
            pltpu.VMEM((ROWS_PER_WORKER, 128), jnp.int32),
            pltpu.VMEM((2 * R, 128, EMBED), jnp.float32),
            pltpu.SemaphoreType.DMA,
            pltpu.SemaphoreType.DMA,
            pltpu.SemaphoreType.DMA,
            pltpu.SemaphoreType.DMA,
            pltpu.SemaphoreType.DMA,
        ],
    )(_sc_body)
    out = gather(x2d, table)
    return out.reshape(x.shape + (EMBED,))


# speedup vs baseline: 1.3118x; 1.2619x over previous
import functools

import jax
import jax.numpy as jnp
from jax import lax
from jax.experimental import pallas as pl
from jax.experimental.pallas import tpu as pltpu
from jax.experimental.pallas import tpu_sc as plsc

EMBED = 128
NUM_WORKERS = 32
TOKENS = 4096 * 200
X_ROWS = TOKENS // 128
ROWS_PER_WORKER = X_ROWS // NUM_WORKERS
R = 2
NCHUNK = ROWS_PER_WORKER // R


def _proj_body(e0, e1, e2, e3, w0, w1, w2, w3, o):
    o[0:100] = jnp.dot(e0[...], w0[...], preferred_element_type=jnp.float32)
    o[100:300] = jnp.dot(e1[...], w1[...], preferred_element_type=jnp.float32)
    o[300:600] = jnp.dot(e2[...], w2[...], preferred_element_type=jnp.float32)
    o[600:1000] = jnp.dot(e3[...], w3[...], preferred_element_type=jnp.float32)


def _project_tables(embs, ws):
    return pl.pallas_call(
        _proj_body,
        out_shape=jax.ShapeDtypeStruct((1000, EMBED), jnp.float32),
    )(*embs, *ws)


NPAIR = NCHUNK // 2


def _sc_body(x_hbm, tab_hbm, out_hbm, tab_sh, idx_v,
             rows_v, gsem0, gsem1, ssem0, ssem1, hsem):
    sid = lax.axis_index("s")
    wid = sid * 2 + lax.axis_index("c")
    row0 = wid * ROWS_PER_WORKER
    gsems = (gsem0, gsem1)
    ssems = (ssem0, ssem1)

    @pl.when(sid == 0)
    def _():
        pltpu.sync_copy(tab_hbm, tab_sh)

    pltpu.sync_copy(x_hbm.at[pl.ds(row0, ROWS_PER_WORKER)], idx_v)
    plsc.subcore_barrier()

    def gissue(jj, b, srcs):
        for t in range(R):
            sem = hsem if srcs[t] is tab_hbm else gsems[b]
            pltpu.async_copy(srcs[t].at[idx_v.at[jj * R + t]],
                             rows_v.at[b * R + t], sem)

    def gwait(b, srcs):
        for t in range(R):
            sem = hsem if srcs[t] is tab_hbm else gsems[b]
            pltpu.make_async_copy(srcs[t].at[idx_v.at[b * R + t]],
                                  rows_v.at[b * R + t], sem).wait()

    def sissue(jj, b):
        pltpu.async_copy(rows_v.at[pl.ds(b * R, R)],
                         out_hbm.at[pl.ds(row0 + jj * R, R)], ssems[b])

    def swait(b):
        pltpu.make_async_copy(rows_v.at[pl.ds(b * R, R)],
                              out_hbm.at[pl.ds(row0, R)], ssems[b]).wait()

    gissue(0, 0, (tab_sh, tab_sh))

    def pair(g, carry):
        jj0 = 2 * g
        gwait(0, (tab_sh, tab_sh))
        sissue(jj0, 0)

        @pl.when(g > 0)
        def _():
            swait(1)

        gissue(jj0 + 1, 1, (tab_sh, tab_sh))

        gwait(1, (tab_sh, tab_sh))
        sissue(jj0 + 1, 1)
        swait(0)

        @pl.when(g < NPAIR - 1)
        def _():
            gissue(jj0 + 2, 0, (tab_sh, tab_sh))

        return carry

    lax.fori_loop(0, NPAIR, pair, 0)
    swait(1)


def kernel(x, emb0, emb1, emb2, emb3, W0, W1, W2, W3):
    table = _project_tables([emb0, emb1, emb2, emb3], [W0, W1, W2, W3])
    x2d = x.reshape(X_ROWS, 128)

    mesh = plsc.VectorSubcoreMesh(core_axis_name="c", subcore_axis_name="s")
    gather = functools.partial(
        pl.kernel,
        mesh=mesh,
        out_type=jax.ShapeDtypeStruct((X_ROWS, 128, EMBED), jnp.float32),
        scratch_types=[
            pltpu.VMEM_SHARED((1000, EMBED), jnp.float32),
            pltpu.VMEM((ROWS_PER_WORKER, 128), jnp.int32),
            pltpu.VMEM((2 * R, 128, EMBED), jnp.float32),
            pltpu.SemaphoreType.DMA,
            pltpu.SemaphoreType.DMA,
            pltpu.SemaphoreType.DMA,
            pltpu.SemaphoreType.DMA,
            pltpu.SemaphoreType.DMA,
        ],
    )(_sc_body)
    out = gather(x2d, table)
    return out.reshape(x.shape + (EMBED,))


# speedup vs baseline: 1.3439x; 1.0244x over previous
import functools

import jax
import jax.numpy as jnp
from jax import lax
from jax.experimental import pallas as pl
from jax.experimental.pallas import tpu as pltpu
from jax.experimental.pallas import tpu_sc as plsc

EMBED = 128
NUM_WORKERS = 32
TOKENS = 4096 * 200
X_ROWS = TOKENS // 128
ROWS_PER_WORKER = X_ROWS // NUM_WORKERS
NBUF = 4
NGROUP = (ROWS_PER_WORKER - 4) // NBUF


def _proj_body(e0, e1, e2, e3, w0, w1, w2, w3, o):
    o[0:100] = jnp.dot(e0[...], w0[...], preferred_element_type=jnp.float32)
    o[100:300] = jnp.dot(e1[...], w1[...], preferred_element_type=jnp.float32)
    o[300:600] = jnp.dot(e2[...], w2[...], preferred_element_type=jnp.float32)
    o[600:1000] = jnp.dot(e3[...], w3[...], preferred_element_type=jnp.float32)


def _project_tables(embs, ws):
    return pl.pallas_call(
        _proj_body,
        out_shape=jax.ShapeDtypeStruct((1000, EMBED), jnp.float32),
    )(*embs, *ws)


def _sc_body(x_hbm, tab_hbm, out_hbm, tab_sh, idx_v, rows_v,
             gsem0, gsem1, gsem2, gsem3, ssem0, ssem1, ssem2, ssem3):
    sid = lax.axis_index("s")
    wid = sid * 2 + lax.axis_index("c")
    row0 = wid * ROWS_PER_WORKER
    gsems = (gsem0, gsem1, gsem2, gsem3)
    ssems = (ssem0, ssem1, ssem2, ssem3)

    @pl.when(sid == 0)
    def _():
        pltpu.sync_copy(tab_hbm, tab_sh)

    pltpu.sync_copy(x_hbm.at[pl.ds(row0, ROWS_PER_WORKER)], idx_v)
    plsc.subcore_barrier()

    def gissue(j, b):
        pltpu.async_copy(tab_sh.at[idx_v.at[j]], rows_v.at[b], gsems[b])

    def gwait(j, b):
        pltpu.make_async_copy(tab_sh.at[idx_v.at[j]],
                              rows_v.at[b], gsems[b]).wait()

    def sissue(j, b):
        pltpu.async_copy(rows_v.at[b], out_hbm.at[row0 + j], ssems[b])

    def swait(b):
        pltpu.make_async_copy(rows_v.at[b], out_hbm.at[row0],
                              ssems[b]).wait()

    gissue(0, 0)
    gissue(1, 1)
    gwait(0, 0)
    sissue(0, 0)
    gissue(2, 2)
    gwait(1, 1)
    sissue(1, 1)
    gissue(3, 3)

    def group(g, carry):
        j0 = 2 + NBUF * g
        for u in range(NBUF):
            b = (2 + u) % NBUF
            bb = (b + 2) % NBUF
            gwait(j0 + u, b)
            sissue(j0 + u, b)
            swait(bb)
            gissue(j0 + u + 2, bb)
        return carry

    lax.fori_loop(0, NGROUP, group, 0)

    gwait(ROWS_PER_WORKER - 2, 2)
    sissue(ROWS_PER_WORKER - 2, 2)
    gwait(ROWS_PER_WORKER - 1, 3)
    sissue(ROWS_PER_WORKER - 1, 3)
    for b in range(NBUF):
        swait(b)


def kernel(x, emb0, emb1, emb2, emb3, W0, W1, W2, W3):
    table = _project_tables([emb0, emb1, emb2, emb3], [W0, W1, W2, W3])
    x2d = x.reshape(X_ROWS, 128)

    mesh = plsc.VectorSubcoreMesh(core_axis_name="c", subcore_axis_name="s")
    gather = functools.partial(
        pl.kernel,
        mesh=mesh,
        out_type=jax.ShapeDtypeStruct((X_ROWS, 128, EMBED), jnp.float32),
        scratch_types=[
            pltpu.VMEM_SHARED((1000, EMBED), jnp.float32),
            pltpu.VMEM((ROWS_PER_WORKER, 128), jnp.int32),
            pltpu.VMEM((NBUF, 128, EMBED), jnp.float32),
            pltpu.SemaphoreType.DMA,
            pltpu.SemaphoreType.DMA,
            pltpu.SemaphoreType.DMA,
            pltpu.SemaphoreType.DMA,
            pltpu.SemaphoreType.DMA,
            pltpu.SemaphoreType.DMA,
            pltpu.SemaphoreType.DMA,
            pltpu.SemaphoreType.DMA,
        ],
    )(_sc_body)
    out = gather(x2d, table)
    return out.reshape(x.shape + (EMBED,))
